# baseline (speedup 1.0000x reference)
import functools

import jax
import jax.numpy as jnp
from jax import lax
from jax.experimental import pallas as pl
from jax.experimental.pallas import tpu as pltpu
from jax.experimental.pallas import tpu_sc as plsc

_C_IN = 1000
_D = 64
_BATCH = 16384
_HIST = 200

_NC = 2
_NS = 16
_NW = _NC * _NS
_L = 16

_B = _BATCH * _HIST
_ROWS_PER_W = _B // _NW
_GROUP = 400
_NGROUPS = _ROWS_PER_W // _GROUP
_NT = _GROUP // _L
_TBL = (_C_IN + 1) * _D


def _sc_body(x_hbm, w_hbm, out_hbm, table_v, idx_v, rows_v,
             sem_t, sem_i0, sem_i1, sem_o0, sem_o1):
    wid = lax.axis_index("s") * _NC + lax.axis_index("c")
    sem_i = (sem_i0, sem_i1)
    sem_o = (sem_o0, sem_o1)

    def start_idx(buf, g):
        pltpu.make_async_copy(x_hbm.at[wid, g], idx_v.at[buf], sem_i[buf]).start()

    def wait_idx(buf):
        pltpu.make_async_copy(x_hbm.at[wid, 0], idx_v.at[buf], sem_i[buf]).wait()

    def start_out(buf, g):
        pltpu.make_async_copy(rows_v.at[buf], out_hbm.at[wid, g], sem_o[buf]).start()

    def wait_out(buf):
        pltpu.make_async_copy(rows_v.at[buf], out_hbm.at[wid, 0], sem_o[buf]).wait()

    pltpu.make_async_copy(w_hbm, table_v, sem_t).start()
    start_idx(0, 0)
    start_idx(1, 1)
    pltpu.make_async_copy(w_hbm, table_v, sem_t).wait()

    lane = lax.iota(jnp.int32, _L)

    def compute(buf):
        rows = rows_v.at[buf]

        def tile(t, carry):
            i16 = idx_v[buf, pl.ds(t * _L, _L)]
            src = i16 * _D
            dst = (t * (_L * _D)) + lane * _D
            for j in range(_D):
                v = plsc.load_gather(table_v, [src + j])
                plsc.store_scatter(rows, [dst + j], v)
            return carry

        lax.fori_loop(0, _NT, tile, 0, unroll=False)

    for buf in (0, 1):
        wait_idx(buf)
        compute(buf)
        start_idx(buf, buf + 2)
        start_out(buf, buf)

    def pair(p, carry):
        g0 = 2 * p
        for buf in (0, 1):
            g = g0 + buf
            wait_out(buf)
            wait_idx(buf)
            compute(buf)
            start_idx(buf, lax.rem(g + 2, _NGROUPS))
            start_out(buf, g)
        return carry

    lax.fori_loop(1, _NGROUPS // 2, pair, 0, unroll=False)

    for buf in (0, 1):
        wait_out(buf)
        wait_idx(buf)


@jax.jit
def _lookup(x32, w_flat):
    mesh = plsc.VectorSubcoreMesh(
        core_axis_name="c", subcore_axis_name="s",
        num_cores=_NC, num_subcores=_NS,
    )
    run = pl.kernel(
        _sc_body,
        out_type=jax.ShapeDtypeStruct((_NW, _NGROUPS, _GROUP * _D), jnp.float32),
        mesh=mesh,
        scratch_types=[
            pltpu.VMEM((_TBL,), jnp.float32),
            pltpu.VMEM((2, _GROUP), jnp.int32),
            pltpu.VMEM((2, _GROUP * _D), jnp.float32),
            pltpu.SemaphoreType.DMA,
            pltpu.SemaphoreType.DMA,
            pltpu.SemaphoreType.DMA,
            pltpu.SemaphoreType.DMA,
            pltpu.SemaphoreType.DMA,
        ],
        compiler_params=pltpu.CompilerParams(
            use_tc_tiling_on_sc=False, needs_layout_passes=False),
    )
    return run(x32, w_flat)


def kernel(x, W):
    x32 = x.reshape(-1).astype(jnp.int32).reshape(_NW, _NGROUPS, _GROUP)
    out = _lookup(x32, W.reshape(-1))
    return lax.stop_gradient(out.reshape(_BATCH, _HIST, _D))


# speedup vs baseline: 2.1388x; 2.1388x over previous
import functools

import jax
import jax.numpy as jnp
from jax import lax
from jax.experimental import pallas as pl
from jax.experimental.pallas import tpu as pltpu
from jax.experimental.pallas import tpu_sc as plsc

_C_IN = 1000
_D = 64
_BATCH = 16384
_HIST = 200

_NC = 2
_NS = 16
_NW = _NC * _NS
_L = 16

_B = _BATCH * _HIST
_ROWS_PER_W = _B // _NW
_GROUP = 400
_NGROUPS = _ROWS_PER_W // _GROUP
_NT = _GROUP // _L
_TBL = (_C_IN + 1) * _D


def _sc_body(x_hbm, w_hbm, out_hbm, table_v, idx_v, rows_v,
             sem_t, sem_i0, sem_i1, sem_o0, sem_o1):
    wid = lax.axis_index("s") * _NC + lax.axis_index("c")
    sem_i = (sem_i0, sem_i1)
    sem_o = (sem_o0, sem_o1)

    def start_idx(buf, g):
        pltpu.make_async_copy(x_hbm.at[wid, g], idx_v.at[buf], sem_i[buf]).start()

    def wait_idx(buf):
        pltpu.make_async_copy(x_hbm.at[wid, 0], idx_v.at[buf], sem_i[buf]).wait()

    def start_out(buf, g):
        pltpu.make_async_copy(rows_v.at[buf], out_hbm.at[wid, g], sem_o[buf]).start()

    def wait_out(buf):
        pltpu.make_async_copy(rows_v.at[buf], out_hbm.at[wid, 0], sem_o[buf]).wait()

    pltpu.make_async_copy(w_hbm, table_v, sem_t).start()
    start_idx(0, 0)
    start_idx(1, 1)
    pltpu.make_async_copy(w_hbm, table_v, sem_t).wait()

    lane = lax.iota(jnp.int32, _L)
    lane_off = lane * _D
    diag = [lax.rem(lane + k, _L) for k in range(_L)]

    def compute(buf):
        rows = rows_v.at[buf]

        def tile(t, carry):
            i16 = idx_v[buf, pl.ds(t * _L, _L)]
            src_row = i16 * _D
            dst_row = (t * (_L * _D)) + lane_off
            for k in range(_L):
                src_k = src_row + diag[k]
                dst_k = dst_row + diag[k]
                for blk in range(0, _D, _L):
                    v = plsc.load_gather(table_v, [src_k + blk])
                    plsc.store_scatter(rows, [dst_k + blk], v)
            return carry

        lax.fori_loop(0, _NT, tile, 0, unroll=False)

    for buf in (0, 1):
        wait_idx(buf)
        compute(buf)
        start_idx(buf, buf + 2)
        start_out(buf, buf)

    def pair(p, carry):
        g0 = 2 * p
        for buf in (0, 1):
            g = g0 + buf
            wait_out(buf)
            wait_idx(buf)
            compute(buf)
            start_idx(buf, lax.rem(g + 2, _NGROUPS))
            start_out(buf, g)
        return carry

    lax.fori_loop(1, _NGROUPS // 2, pair, 0, unroll=False)

    for buf in (0, 1):
        wait_out(buf)
        wait_idx(buf)


@jax.jit
def _lookup(x32, w_flat):
    mesh = plsc.VectorSubcoreMesh(
        core_axis_name="c", subcore_axis_name="s",
        num_cores=_NC, num_subcores=_NS,
    )
    run = pl.kernel(
        _sc_body,
        out_type=jax.ShapeDtypeStruct((_NW, _NGROUPS, _GROUP * _D), jnp.float32),
        mesh=mesh,
        scratch_types=[
            pltpu.VMEM((_TBL,), jnp.float32),
            pltpu.VMEM((2, _GROUP), jnp.int32),
            pltpu.VMEM((2, _GROUP * _D), jnp.float32),
            pltpu.SemaphoreType.DMA,
            pltpu.SemaphoreType.DMA,
            pltpu.SemaphoreType.DMA,
            pltpu.SemaphoreType.DMA,
            pltpu.SemaphoreType.DMA,
        ],
        compiler_params=pltpu.CompilerParams(
            use_tc_tiling_on_sc=False, needs_layout_passes=False),
    )
    return run(x32, w_flat)


def kernel(x, W):
    x32 = x.reshape(-1).astype(jnp.int32).reshape(_NW, _NGROUPS, _GROUP)
    out = _lookup(x32, W.reshape(-1))
    return lax.stop_gradient(out.reshape(_BATCH, _HIST, _D))


# speedup vs baseline: 2.9266x; 1.3683x over previous
import functools

import jax
import jax.numpy as jnp
from jax import lax
from jax.experimental import pallas as pl
from jax.experimental.pallas import tpu as pltpu
from jax.experimental.pallas import tpu_sc as plsc

_C_IN = 1000
_D = 64
_BATCH = 16384
_HIST = 200

_NC = 2
_NS = 16
_NW = _NC * _NS

_B = _BATCH * _HIST
_ROWS_PER_W = _B // _NW
_GATHER = 128
_KPG = 2
_GROUP = _GATHER * _KPG
_NGROUPS = _ROWS_PER_W // _GROUP
_NBUF = 4


def _sc_body(x_hbm, w_hbm, out_hbm, idx_v, rows_v, sems):
    wid = lax.axis_index("s") * _NC + lax.axis_index("c")
    sem_i = [sems.at[0, b] for b in range(_NBUF)]
    sem_g = [sems.at[1, b] for b in range(_NBUF)]
    sem_o = [sems.at[2, b] for b in range(_NBUF)]

    def start_idx(buf, g):
        pltpu.make_async_copy(x_hbm.at[wid, g], idx_v.at[buf], sem_i[buf]).start()

    def wait_idx(buf):
        pltpu.make_async_copy(x_hbm.at[wid, 0], idx_v.at[buf], sem_i[buf]).wait()

    def start_gathers(buf):
        for k in range(_KPG):
            pltpu.make_async_copy(
                w_hbm.at[idx_v.at[buf, k]],
                rows_v.at[buf, pl.ds(k * _GATHER, _GATHER)],
                sem_g[buf],
            ).start()

    def wait_gathers(buf):
        for k in range(_KPG):
            pltpu.make_async_copy(
                w_hbm.at[idx_v.at[buf, k]],
                rows_v.at[buf, pl.ds(k * _GATHER, _GATHER)],
                sem_g[buf],
            ).wait()

    def start_out(buf, g):
        pltpu.make_async_copy(rows_v.at[buf], out_hbm.at[wid, g], sem_o[buf]).start()

    def wait_out(buf):
        pltpu.make_async_copy(rows_v.at[buf], out_hbm.at[wid, 0], sem_o[buf]).wait()

    for b in range(_NBUF):
        start_idx(b, b)
    wait_idx(0)
    start_gathers(0)
    for g in range(1, _NBUF):
        wait_idx(g)
        start_gathers(g)
        b1 = g - 1
        wait_gathers(b1)
        start_idx(b1, b1 + _NBUF)
        start_out(b1, b1)

    def quad(q, carry):
        g0 = _NBUF * q
        for r in range(_NBUF):
            b = r
            g = g0 + r
            wait_out(b)
            wait_idx(b)
            start_gathers(b)
            b1 = (r - 1) % _NBUF
            wait_gathers(b1)
            start_idx(b1, lax.rem(g - 1 + _NBUF, _NGROUPS))
            start_out(b1, g - 1)
        return carry

    lax.fori_loop(1, _NGROUPS // _NBUF, quad, 0, unroll=False)

    last_b = (_NGROUPS - 1) % _NBUF
    wait_gathers(last_b)
    start_out(last_b, _NGROUPS - 1)
    for b in range(_NBUF):
        wait_out(b)
    for b in range(_NBUF - 1):
        wait_idx(b)


@jax.jit
def _lookup(x32, w):
    mesh = plsc.VectorSubcoreMesh(
        core_axis_name="c", subcore_axis_name="s",
        num_cores=_NC, num_subcores=_NS,
    )
    run = pl.kernel(
        _sc_body,
        out_type=jax.ShapeDtypeStruct((_NW, _NGROUPS, _GROUP, _D), jnp.float32),
        mesh=mesh,
        scratch_types=[
            pltpu.VMEM((_NBUF, _KPG, _GATHER), jnp.int32),
            pltpu.VMEM((_NBUF, _GROUP, _D), jnp.float32),
            pltpu.SemaphoreType.DMA((3, _NBUF)),
        ],
        compiler_params=pltpu.CompilerParams(use_tc_tiling_on_sc=False),
    )
    return run(x32, w)


def kernel(x, W):
    x32 = x.reshape(-1).astype(jnp.int32).reshape(_NW, _NGROUPS, _KPG, _GATHER)
    out = _lookup(x32, W)
    return lax.stop_gradient(out.reshape(_BATCH, _HIST, _D))
